# baseline (speedup 1.0000x reference)
import jax
import jax.numpy as jnp
from jax.experimental import pallas as pl
from jax.experimental.pallas import tpu as pltpu

_B, _Q, _C, _V = 4, 20000, 81, 117
_BQ = 2000
_NQ = _Q // _BQ
_SUBJECT_CATEGORY_ID = 0


def _fused_body(obj_ref, verb_ref, sub_ref, objb_ref, scale_ref,
                labels_ref, scores_ref, vs_ref, subxy_ref, objxy_ref):
    lg = obj_ref[0]
    m = jnp.max(lg, axis=-1, keepdims=True)
    e = jnp.exp(lg - m)
    s = jnp.sum(e, axis=-1)

    lg80 = lg[:, : _C - 1]
    m80 = jnp.max(lg80, axis=-1)
    score = jnp.exp(m80 - m[:, 0]) / s

    ids = jax.lax.broadcasted_iota(jnp.int32, lg80.shape, 1)
    lab = jnp.min(jnp.where(lg80 == m80[:, None], ids, _C - 1), axis=-1)

    labels_ref[0, 0] = lab
    scores_ref[0, 0] = score

    vb = verb_ref[0]
    sig = 1.0 / (1.0 + jnp.exp(-vb))
    vs_ref[0] = sig * score[:, None]

    sc = scale_ref[0]
    for bref, oref in ((sub_ref, subxy_ref), (objb_ref, objxy_ref)):
        b = bref[0]
        xy = b[:, 0:2]
        wh = 0.5 * b[:, 2:4]
        oref[0] = jnp.concatenate([xy - wh, xy + wh], axis=1) * sc


def _postprocess(pred_obj_logits, pred_verb_logits, pred_sub_boxes,
                 pred_obj_boxes, scale_fct):
    grid = (_B, _NQ)
    out_shapes = (
        jax.ShapeDtypeStruct((_B * _NQ, 1, _BQ), jnp.int32),
        jax.ShapeDtypeStruct((_B * _NQ, 1, _BQ), jnp.float32),
        jax.ShapeDtypeStruct((_B, _Q, _V), jnp.float32),
        jax.ShapeDtypeStruct((_B, _Q, 4), jnp.float32),
        jax.ShapeDtypeStruct((_B, _Q, 4), jnp.float32),
    )
    in_specs = [
        pl.BlockSpec((1, _BQ, _C), lambda b, q: (b, q, 0)),
        pl.BlockSpec((1, _BQ, _V), lambda b, q: (b, q, 0)),
        pl.BlockSpec((1, _BQ, 4), lambda b, q: (b, q, 0)),
        pl.BlockSpec((1, _BQ, 4), lambda b, q: (b, q, 0)),
        pl.BlockSpec((1, 1, 4), lambda b, q: (b, 0, 0)),
    ]
    out_specs = (
        pl.BlockSpec((1, 1, _BQ), lambda b, q: (b * _NQ + q, 0, 0)),
        pl.BlockSpec((1, 1, _BQ), lambda b, q: (b * _NQ + q, 0, 0)),
        pl.BlockSpec((1, _BQ, _V), lambda b, q: (b, q, 0)),
        pl.BlockSpec((1, _BQ, 4), lambda b, q: (b, q, 0)),
        pl.BlockSpec((1, _BQ, 4), lambda b, q: (b, q, 0)),
    )
    return pl.pallas_call(
        _fused_body,
        grid=grid,
        in_specs=in_specs,
        out_specs=out_specs,
        out_shape=out_shapes,
        compiler_params=pltpu.CompilerParams(
            dimension_semantics=("parallel", "parallel"),
        ),
    )(pred_obj_logits, pred_verb_logits, pred_sub_boxes, pred_obj_boxes,
      scale_fct.reshape(_B, 1, 4))


def kernel(pred_obj_logits, pred_verb_logits, pred_sub_boxes, pred_obj_boxes, target_sizes):
    img_h = target_sizes[:, 0].astype(jnp.float32)
    img_w = target_sizes[:, 1].astype(jnp.float32)
    scale_fct = jnp.stack([img_w, img_h, img_w, img_h], axis=1)

    labels3, scores3, vs, sub_xy, obj_xy = _postprocess(
        pred_obj_logits, pred_verb_logits, pred_sub_boxes, pred_obj_boxes,
        scale_fct)

    obj_labels = labels3.reshape(_B, _Q)
    obj_scores = scores3.reshape(_B, _Q)
    sl = jnp.full_like(obj_labels, _SUBJECT_CATEGORY_ID)
    labels = jnp.concatenate([sl, obj_labels], axis=1)
    boxes = jnp.concatenate([sub_xy, obj_xy], axis=1)

    ids = jnp.arange(2 * _Q)
    sub_ids = ids[:_Q]
    obj_ids = ids[_Q:]

    return (labels, boxes, vs, pred_verb_logits, sub_ids, obj_ids, obj_scores)
